# baseline (speedup 1.0000x reference)
import functools

import jax
import jax.numpy as jnp
from jax import lax
from jax.experimental import pallas as pl
from jax.experimental.pallas import tpu as pltpu
from jax.experimental.pallas import tpu_sc as plsc

_NUM_CORES = 2
_NUM_SUBCORES = 16
_NW = _NUM_CORES * _NUM_SUBCORES


@functools.cache
def _make_lookup(S, T, D):
    slab = S // _NW
    assert S % _NW == 0 and slab % 16 == 0
    mesh = plsc.VectorSubcoreMesh(core_axis_name="c", subcore_axis_name="s")

    @functools.partial(
        pl.kernel,
        out_type=jax.ShapeDtypeStruct((T, D, S), jnp.float32),
        mesh=mesh,
        scratch_types=[
            pltpu.VMEM((2, slab), jnp.int32),
            pltpu.VMEM((2, slab, D), jnp.float32),
            pltpu.VMEM((2, D, slab), jnp.float32),
            pltpu.SemaphoreType.DMA,
            pltpu.SemaphoreType.DMA,
            pltpu.SemaphoreType.DMA,
        ],
        compiler_params=pltpu.CompilerParams(
            use_tc_tiling_on_sc=False, needs_layout_passes=False
        ),
    )
    def k(tokt_hbm, table_hbm, out_hbm, idx_v, rows_v, tr_v, sem_i, sem_g, sem_w):
        wid = lax.axis_index("s") * _NUM_CORES + lax.axis_index("c")
        b0 = wid * slab
        iota16 = lax.iota(jnp.int32, 16)

        def idx_src(j):
            return tokt_hbm.at[j, pl.ds(b0, slab)]

        def out_dst(j):
            return out_hbm.at[j, :, pl.ds(b0, slab)]

        cvecs = [jnp.full((16,), d, jnp.int32) for d in range(D)]

        def transpose(b):
            src = rows_v.at[b]
            dst = tr_v.at[b]

            @plsc.parallel_loop(0, slab // 16, 1, unroll=8)
            def _(m):
                ridx = iota16 + 16 * m
                for d in range(D):
                    v = plsc.load_gather(src, [ridx, cvecs[d]])
                    dst[d, pl.ds(16 * m, 16)] = v

        pltpu.async_copy(idx_src(0), idx_v.at[0], sem_i)
        pltpu.make_async_copy(idx_src(0), idx_v.at[0], sem_i).wait()
        pltpu.async_copy(table_hbm.at[idx_v.at[0]], rows_v.at[0], sem_g)
        pltpu.async_copy(idx_src(1), idx_v.at[1], sem_i)

        def step(j, carry):
            b = j % 2
            pltpu.make_async_copy(
                table_hbm.at[idx_v.at[b]], rows_v.at[b], sem_g
            ).wait()

            @pl.when(j + 1 < T)
            def _():
                nb = (j + 1) % 2
                pltpu.make_async_copy(idx_src(0), idx_v.at[nb], sem_i).wait()
                pltpu.async_copy(
                    table_hbm.at[idx_v.at[nb]], rows_v.at[nb], sem_g
                )

                @pl.when(j + 2 < T)
                def _():
                    pltpu.async_copy(idx_src(j + 2), idx_v.at[b], sem_i)

            @pl.when(j >= 2)
            def _():
                pltpu.make_async_copy(tr_v.at[b], out_dst(0), sem_w).wait()

            transpose(b)
            pltpu.async_copy(tr_v.at[b], out_dst(j), sem_w)
            return carry

        lax.fori_loop(0, T, step, 0)
        pltpu.make_async_copy(tr_v.at[0], out_dst(0), sem_w).wait()
        pltpu.make_async_copy(tr_v.at[1], out_dst(0), sem_w).wait()

    return k


def kernel(token_ids, weight):
    S, T = token_ids.shape
    _, D = weight.shape
    tokt = jnp.transpose(token_ids).astype(jnp.int32)
    o_phys = _make_lookup(S, T, D)(tokt, weight)
    return jnp.transpose(o_phys, (2, 0, 1))


# speedup vs baseline: 2.1354x; 2.1354x over previous
import functools

import jax
import jax.numpy as jnp
from jax import lax
from jax.experimental import pallas as pl
from jax.experimental.pallas import tpu as pltpu
from jax.experimental.pallas import tpu_sc as plsc

_NUM_CORES = 2
_NUM_SUBCORES = 16
_NW = _NUM_CORES * _NUM_SUBCORES


@functools.cache
def _make_lookup(S, T, D):
    slab = S // _NW
    assert S % _NW == 0 and slab % 16 == 0
    mesh = plsc.VectorSubcoreMesh(core_axis_name="c", subcore_axis_name="s")

    @functools.partial(
        pl.kernel,
        out_type=jax.ShapeDtypeStruct((T, D, S), jnp.float32),
        mesh=mesh,
        scratch_types=[
            pltpu.VMEM((2, slab), jnp.int32),
            pltpu.VMEM((2, slab, D), jnp.float32),
            pltpu.VMEM((2, D, slab + 1), jnp.float32),
            pltpu.SemaphoreType.DMA,
            pltpu.SemaphoreType.DMA,
            pltpu.SemaphoreType.DMA,
        ],
        compiler_params=pltpu.CompilerParams(
            use_tc_tiling_on_sc=False, needs_layout_passes=False
        ),
    )
    def k(tokt_hbm, table_hbm, out_hbm, idx_v, rows_v, tr_v, sem_i, sem_g, sem_w):
        wid = lax.axis_index("s") * _NUM_CORES + lax.axis_index("c")
        b0 = wid * slab
        iota16 = lax.iota(jnp.int32, 16)

        def idx_src(j):
            return tokt_hbm.at[j, pl.ds(b0, slab)]

        def out_dst(j):
            return out_hbm.at[j, :, pl.ds(b0, slab)]

        iota16p16 = iota16 + 16

        def transpose(b):
            src = rows_v.at[b]
            dst = tr_v.at[b]

            @plsc.parallel_loop(0, slab, 1, unroll=4)
            def _(kk):
                kvec = jnp.zeros((16,), jnp.int32) + kk
                v0 = src[kk, pl.ds(0, 16)]
                v1 = src[kk, pl.ds(16, 16)]
                plsc.store_scatter(dst, [iota16, kvec], v0)
                plsc.store_scatter(dst, [iota16p16, kvec], v1)

        pltpu.async_copy(idx_src(0), idx_v.at[0], sem_i)
        pltpu.make_async_copy(idx_src(0), idx_v.at[0], sem_i).wait()
        pltpu.async_copy(table_hbm.at[idx_v.at[0]], rows_v.at[0], sem_g)
        pltpu.async_copy(idx_src(1), idx_v.at[1], sem_i)

        def step(j, carry):
            b = j % 2
            pltpu.make_async_copy(
                table_hbm.at[idx_v.at[b]], rows_v.at[b], sem_g
            ).wait()

            @pl.when(j + 1 < T)
            def _():
                nb = (j + 1) % 2
                pltpu.make_async_copy(idx_src(0), idx_v.at[nb], sem_i).wait()
                pltpu.async_copy(
                    table_hbm.at[idx_v.at[nb]], rows_v.at[nb], sem_g
                )

                @pl.when(j + 2 < T)
                def _():
                    pltpu.async_copy(idx_src(j + 2), idx_v.at[b], sem_i)

            @pl.when(j >= 2)
            def _():
                pltpu.make_async_copy(tr_v.at[b, :, pl.ds(0, slab)], out_dst(0), sem_w).wait()

            transpose(b)
            pltpu.async_copy(tr_v.at[b, :, pl.ds(0, slab)], out_dst(j), sem_w)
            return carry

        lax.fori_loop(0, T, step, 0)
        pltpu.make_async_copy(tr_v.at[0, :, pl.ds(0, slab)], out_dst(0), sem_w).wait()
        pltpu.make_async_copy(tr_v.at[1, :, pl.ds(0, slab)], out_dst(0), sem_w).wait()

    return k


def kernel(token_ids, weight):
    S, T = token_ids.shape
    _, D = weight.shape
    tokt = jnp.transpose(token_ids).astype(jnp.int32)
    o_phys = _make_lookup(S, T, D)(tokt, weight)
    return jnp.transpose(o_phys, (2, 0, 1))


# speedup vs baseline: 3.2510x; 1.5224x over previous
import functools

import jax
import jax.numpy as jnp
from jax import lax
from jax.experimental import pallas as pl
from jax.experimental.pallas import tpu as pltpu
from jax.experimental.pallas import tpu_sc as plsc

_NUM_CORES = 2
_NUM_SUBCORES = 16
_NW = _NUM_CORES * _NUM_SUBCORES


@functools.cache
def _make_lookup(S, T, D):
    slab = S // _NW
    assert S % _NW == 0 and slab % 16 == 0
    mesh = plsc.VectorSubcoreMesh(core_axis_name="c", subcore_axis_name="s")

    @functools.partial(
        pl.kernel,
        out_type=jax.ShapeDtypeStruct((T, D // 8, 128, 8, 128), jnp.float32),
        mesh=mesh,
        scratch_types=[
            pltpu.VMEM((2, slab), jnp.int32),
            pltpu.VMEM((2, slab, D), jnp.float32),
            pltpu.VMEM((2, D // 8, 5, 8, 131), jnp.float32),
            pltpu.SemaphoreType.DMA,
            pltpu.SemaphoreType.DMA,
            pltpu.SemaphoreType.DMA,
        ],
        compiler_params=pltpu.CompilerParams(
            use_tc_tiling_on_sc=False, needs_layout_passes=False
        ),
    )
    def k(tokt_hbm, table_hbm, out_hbm, idx_v, rows_v, tr_v, sem_i, sem_g, sem_w):
        wid = lax.axis_index("s") * _NUM_CORES + lax.axis_index("c")
        b0 = wid * slab
        iota16 = lax.iota(jnp.int32, 16)

        def idx_src(j):
            return tokt_hbm.at[j, pl.ds(b0, slab)]

        nc = slab // 128
        c0 = wid * nc

        def out_dst(j):
            return out_hbm.at[j, :, pl.ds(c0, nc), :, :]

        def tr_src(b):
            return tr_v.at[b, :, pl.ds(0, nc), :, pl.ds(0, 128)]

        r_v0 = iota16 // 8
        r_v1 = r_v0 + 2
        rr_v = lax.rem(iota16, 8)

        def transpose(b):
            src = rows_v.at[b]
            dst = tr_v.at[b]

            @plsc.parallel_loop(0, slab, 1, unroll=4)
            def _(kk):
                cl = jnp.zeros((16,), jnp.int32) + kk // 128
                cc = jnp.zeros((16,), jnp.int32) + lax.rem(kk, 128)
                v0 = src[kk, pl.ds(0, 16)]
                v1 = src[kk, pl.ds(16, 16)]
                plsc.store_scatter(dst, [r_v0, cl, rr_v, cc], v0)
                plsc.store_scatter(dst, [r_v1, cl, rr_v, cc], v1)

        pltpu.async_copy(idx_src(0), idx_v.at[0], sem_i)
        pltpu.make_async_copy(idx_src(0), idx_v.at[0], sem_i).wait()
        pltpu.async_copy(table_hbm.at[idx_v.at[0]], rows_v.at[0], sem_g)
        pltpu.async_copy(idx_src(1), idx_v.at[1], sem_i)

        def step(j, carry):
            b = j % 2
            pltpu.make_async_copy(
                table_hbm.at[idx_v.at[b]], rows_v.at[b], sem_g
            ).wait()

            @pl.when(j + 1 < T)
            def _():
                nb = (j + 1) % 2
                pltpu.make_async_copy(idx_src(0), idx_v.at[nb], sem_i).wait()
                pltpu.async_copy(
                    table_hbm.at[idx_v.at[nb]], rows_v.at[nb], sem_g
                )

                @pl.when(j + 2 < T)
                def _():
                    pltpu.async_copy(idx_src(j + 2), idx_v.at[b], sem_i)

            @pl.when(j >= 2)
            def _():
                pltpu.make_async_copy(tr_src(b), out_dst(0), sem_w).wait()

            transpose(b)
            pltpu.async_copy(tr_src(b), out_dst(j), sem_w)
            return carry

        lax.fori_loop(0, T, step, 0)
        pltpu.make_async_copy(tr_src(0), out_dst(0), sem_w).wait()
        pltpu.make_async_copy(tr_src(1), out_dst(0), sem_w).wait()

    return k


def kernel(token_ids, weight):
    S, T = token_ids.shape
    _, D = weight.shape
    tokt = jnp.transpose(token_ids).astype(jnp.int32)
    o5 = _make_lookup(S, T, D)(tokt, weight)
    return jnp.transpose(o5, (2, 4, 0, 1, 3)).reshape(S, T, D)
